# baseline (speedup 1.0000x reference)
import functools

import jax
import jax.numpy as jnp
from jax import lax
from jax.experimental import pallas as pl
from jax.experimental.pallas import tpu as pltpu
from jax.experimental.pallas import tpu_sc as plsc

B, T, C = 8, 1024, 256
H = 128
K = 32
BIG = 1e9

NC, NS, L = 2, 16, 16
NW = NC * NS
ROWS = B * T
ROWS_PER_W = ROWS // NW
CHUNK = 32
VPR = T // L
IL = 8


def _dist_kernel(x_ref, wp_ref, bp_ref, h_ref, d2_ref):
    x = x_ref[0]
    h = lax.dot_general(
        x, wp_ref[...], (((1,), (1,)), ((), ())),
        preferred_element_type=jnp.float32,
    ) + bp_ref[...]
    h_ref[0] = h
    sq = jnp.sum(h * h, axis=1, keepdims=True)
    g = lax.dot_general(
        h, h, (((1,), (1,)), ((), ())),
        preferred_element_type=jnp.float32,
    )
    d2 = jnp.maximum(sq + jnp.transpose(sq) - 2.0 * g, 0.0)
    row = lax.broadcasted_iota(jnp.int32, (T, T), 0)
    col = lax.broadcasted_iota(jnp.int32, (T, T), 1)
    d2_ref[0] = jnp.where(row == col, BIG, d2)


def _distances(x, W_proj, b_proj):
    nb = x.shape[0]
    return pl.pallas_call(
        _dist_kernel,
        grid=(nb,),
        in_specs=[
            pl.BlockSpec((1, T, C), lambda b: (b, 0, 0)),
            pl.BlockSpec((H, C), lambda b: (0, 0)),
            pl.BlockSpec((1, H), lambda b: (0, 0)),
        ],
        out_specs=[
            pl.BlockSpec((1, T, H), lambda b: (b, 0, 0)),
            pl.BlockSpec((1, T, T), lambda b: (b, 0, 0)),
        ],
        out_shape=[
            jax.ShapeDtypeStruct((nb, T, H), jnp.float32),
            jax.ShapeDtypeStruct((nb, T, T), jnp.float32),
        ],
    )(x, W_proj, b_proj.reshape(1, H))


def _sort_asc(v):
    return plsc.sort_key_val(v, v)[0]


def _sort_desc(v):
    return plsc.sort_key_val(v, v, descending=True)[0]


def _init32(load):
    a = _sort_asc(load(0))
    bd = _sort_desc(load(1))
    lo = jnp.minimum(a, bd)
    hi = jnp.maximum(a, bd)
    return _sort_asc(lo), _sort_asc(hi)


def _step32(t0, t1, load, p):
    a = _sort_asc(load(2 * p))
    bd = _sort_desc(load(2 * p + 1))
    lo = jnp.minimum(a, bd)
    hi = jnp.maximum(a, bd)
    sd0 = _sort_desc(hi)
    sd1 = _sort_desc(lo)
    c0 = jnp.minimum(t0, sd0)
    c1 = jnp.minimum(t1, sd1)
    lo = jnp.minimum(c0, c1)
    hi = jnp.maximum(c0, c1)
    return _sort_asc(lo), _sort_asc(hi)


INF = 3.0e38


def _pivot(load):
    me = load(0)
    mo = load(1)
    for j in range(1, VPR // 2):
        me = jnp.minimum(me, load(2 * j))
        mo = jnp.minimum(mo, load(2 * j + 1))
    return jnp.max(jnp.maximum(me, mo))


def _survivor_thresholds(surv, cnt):
    nil = len(cnt)
    svlen = T + 2 * L
    loads = [(lambda q, rr=rr: surv[pl.ds(rr * svlen + q * L, L)])
             for rr in range(nil)]
    ts = [_init32(ld) for ld in loads]
    nst = [(c + 31) // 32 for c in cnt]
    nmax = nst[0]
    for c in nst[1:]:
        nmax = jnp.maximum(nmax, c)

    def tstep(p, state):
        out = []
        for rr in range(nil):
            t0, t1 = state[rr]
            n0, n1 = _step32(t0, t1, loads[rr], p)
            live = p < nst[rr]
            out.append((jnp.where(live, n0, t0), jnp.where(live, n1, t1)))
        return tuple(out)

    ts = lax.fori_loop(1, nmax, tstep, tuple(ts))
    return [jnp.max(t1) for (_, t1) in ts]


def _topk_kernel(rows_per_w, d2_hbm, thr_hbm, buf0, surv, thrbuf):
    wid = lax.axis_index("s") * NC + lax.axis_index("c")
    base = wid * rows_per_w
    mask0 = lax.iota(jnp.int32, L) == 0
    inf_vec = jnp.full((L,), INF, jnp.float32)
    buf = buf0
    nch = rows_per_w // CHUNK

    def chunk_body(c, carry):
        pltpu.sync_copy(d2_hbm.at[pl.ds(base + c * CHUNK, CHUNK)], buf)

        def rows_body(j, carry2):
            loads = [
                (lambda q, r=j * IL + rr: buf[r, pl.ds(q * L, L)])
                for rr in range(IL)
            ]
            pivots = [_pivot(ld) for ld in loads]
            cnt = [jnp.int32(0)] * IL
            for q in range(0, VPR, 2):
                vs = [(loads[rr](q), loads[rr](q + 1)) for rr in range(IL)]
                ms = [(va <= pivots[rr], vb <= pivots[rr])
                      for rr, (va, vb) in enumerate(vs)]
                cs = [(plsc.all_reduce_population_count(ma)[0],
                       plsc.all_reduce_population_count(mb)[0])
                      for (ma, mb) in ms]
                for rr in range(IL):
                    base_rr = rr * (T + 2 * L)
                    plsc.store_compressed(
                        surv.at[pl.ds(base_rr + cnt[rr], L)],
                        vs[rr][0], mask=ms[rr][0])
                    plsc.store_compressed(
                        surv.at[pl.ds(base_rr + cnt[rr] + cs[rr][0], L)],
                        vs[rr][1], mask=ms[rr][1])
                    cnt[rr] = cnt[rr] + cs[rr][0] + cs[rr][1]
            for rr in range(IL):
                base_rr = rr * (T + 2 * L)
                surv[pl.ds(base_rr + cnt[rr], L)] = inf_vec
                surv[pl.ds(base_rr + cnt[rr] + L, L)] = inf_vec
            thrs = _survivor_thresholds(surv, cnt)
            for rr in range(IL):
                plsc.store_scatter(
                    thrbuf,
                    [jnp.full((L,), c * CHUNK + j * IL + rr, jnp.int32)],
                    jnp.full((L,), thrs[rr], jnp.float32),
                    mask=mask0,
                )
            return carry2

        return lax.fori_loop(0, CHUNK // IL, rows_body, carry)

    lax.fori_loop(0, nch, chunk_body, 0)
    pltpu.sync_copy(thrbuf, thr_hbm.at[pl.ds(base, rows_per_w)])


def _thresholds(d2_flat):
    rows = d2_flat.shape[0]
    rows_per_w = rows // NW
    f = pl.kernel(
        functools.partial(_topk_kernel, rows_per_w),
        out_type=jax.ShapeDtypeStruct((rows,), jnp.float32),
        mesh=plsc.VectorSubcoreMesh(
            core_axis_name="c", subcore_axis_name="s",
            num_cores=NC, num_subcores=NS),
        scratch_types=[
            pltpu.VMEM((CHUNK, T), jnp.float32),
            pltpu.VMEM((IL * (T + 2 * L),), jnp.float32),
            pltpu.VMEM((rows_per_w,), jnp.float32),
        ],
        compiler_params=pltpu.CompilerParams(needs_layout_passes=False),
    )
    return f(d2_flat)


def _smooth_kernel(h_ref, thr_ref, wo_ref, bo_ref, out_ref):
    h = h_ref[0]
    thr = thr_ref[0]
    sq = jnp.sum(h * h, axis=1, keepdims=True)
    g = lax.dot_general(
        h, h, (((1,), (1,)), ((), ())),
        preferred_element_type=jnp.float32,
    )
    d2 = jnp.maximum(sq + jnp.transpose(sq) - 2.0 * g, 0.0)
    row = lax.broadcasted_iota(jnp.int32, (T, T), 0)
    col = lax.broadcasted_iota(jnp.int32, (T, T), 1)
    d2 = jnp.where(row == col, BIG, d2)
    w = jnp.where(d2 <= thr, jnp.exp(d2 * (-1.0 / (2.0 + 1e-8))), 0.0)
    z = jnp.sum(w, axis=1, keepdims=True) + 1e-8
    smooth = lax.dot_general(
        w / z, h, (((1,), (0,)), ((), ())),
        preferred_element_type=jnp.float32,
    )
    lap = jnp.tanh(h - smooth)
    out_ref[0] = lax.dot_general(
        lap, wo_ref[...], (((1,), (1,)), ((), ())),
        preferred_element_type=jnp.float32,
    ) + bo_ref[...]


def _smooth(h, thr, W_out, b_out):
    nb = h.shape[0]
    return pl.pallas_call(
        _smooth_kernel,
        grid=(nb,),
        in_specs=[
            pl.BlockSpec((1, T, H), lambda b: (b, 0, 0)),
            pl.BlockSpec((1, T, 1), lambda b: (b, 0, 0)),
            pl.BlockSpec((H, H), lambda b: (0, 0)),
            pl.BlockSpec((1, H), lambda b: (0, 0)),
        ],
        out_specs=pl.BlockSpec((1, T, H), lambda b: (b, 0, 0)),
        out_shape=jax.ShapeDtypeStruct((nb, T, H), jnp.float32),
    )(h, thr, W_out, b_out.reshape(1, H))


NB_G = 8


@jax.jit
def kernel(x, W_proj, b_proj, W_out, b_out):
    outs = []
    for g in range(B // NB_G):
        xg = x[g * NB_G:(g + 1) * NB_G]
        h, d2 = _distances(xg, W_proj, b_proj)
        thr = _thresholds(d2.reshape(NB_G * T, T))
        outs.append(_smooth(h, thr.reshape(NB_G, T, 1), W_out, b_out))
    return jnp.concatenate(outs, axis=0)


# speedup vs baseline: 1.2058x; 1.2058x over previous
import functools

import jax
import jax.numpy as jnp
from jax import lax
from jax.experimental import pallas as pl
from jax.experimental.pallas import tpu as pltpu
from jax.experimental.pallas import tpu_sc as plsc

B, T, C = 8, 1024, 256
H = 128
K = 32
BIG = 1e9

NC, NS, L = 2, 16, 16
NW = NC * NS
ROWS = B * T
ROWS_PER_W = ROWS // NW
CHUNK = 64
VPR = T // L
IL = 4


def _dist_kernel(x_ref, wp_ref, bp_ref, h_ref, d2_ref):
    x = x_ref[0]
    h = lax.dot_general(
        x, wp_ref[...], (((1,), (1,)), ((), ())),
        preferred_element_type=jnp.float32,
    ) + bp_ref[...]
    h_ref[0] = h
    sq = jnp.sum(h * h, axis=1, keepdims=True)
    g = lax.dot_general(
        h, h, (((1,), (1,)), ((), ())),
        preferred_element_type=jnp.float32,
    )
    d2 = jnp.maximum(sq + jnp.transpose(sq) - 2.0 * g, 0.0)
    row = lax.broadcasted_iota(jnp.int32, (T, T), 0)
    col = lax.broadcasted_iota(jnp.int32, (T, T), 1)
    d2_ref[0] = jnp.where(row == col, BIG, d2)


def _distances(x, W_proj, b_proj):
    nb = x.shape[0]
    return pl.pallas_call(
        _dist_kernel,
        grid=(nb,),
        in_specs=[
            pl.BlockSpec((1, T, C), lambda b: (b, 0, 0)),
            pl.BlockSpec((H, C), lambda b: (0, 0)),
            pl.BlockSpec((1, H), lambda b: (0, 0)),
        ],
        out_specs=[
            pl.BlockSpec((1, T, H), lambda b: (b, 0, 0)),
            pl.BlockSpec((1, T, T), lambda b: (b, 0, 0)),
        ],
        out_shape=[
            jax.ShapeDtypeStruct((nb, T, H), jnp.float32),
            jax.ShapeDtypeStruct((nb, T, T), jnp.float32),
        ],
    )(x, W_proj, b_proj.reshape(1, H))


def _sort_asc(v):
    return plsc.sort_key_val(v, v)[0]


def _sort_desc(v):
    return plsc.sort_key_val(v, v, descending=True)[0]


def _init32(load):
    a = _sort_asc(load(0))
    bd = _sort_desc(load(1))
    lo = jnp.minimum(a, bd)
    hi = jnp.maximum(a, bd)
    return _sort_asc(lo), _sort_asc(hi)


def _step32(t0, t1, load, p):
    a = _sort_asc(load(2 * p))
    bd = _sort_desc(load(2 * p + 1))
    lo = jnp.minimum(a, bd)
    hi = jnp.maximum(a, bd)
    sd0 = _sort_desc(hi)
    sd1 = _sort_desc(lo)
    c0 = jnp.minimum(t0, sd0)
    c1 = jnp.minimum(t1, sd1)
    lo = jnp.minimum(c0, c1)
    hi = jnp.maximum(c0, c1)
    return _sort_asc(lo), _sort_asc(hi)


INF = 3.0e38


def _pivot(load):
    me = load(0)
    mo = load(1)
    for j in range(1, VPR // 2):
        me = jnp.minimum(me, load(2 * j))
        mo = jnp.minimum(mo, load(2 * j + 1))
    return jnp.max(jnp.maximum(me, mo))


def _survivor_thresholds(surv, cnt):
    nil = len(cnt)
    svlen = T + 2 * L
    loads = [(lambda q, rr=rr: surv[pl.ds(rr * svlen + q * L, L)])
             for rr in range(nil)]
    ts = [_init32(ld) for ld in loads]
    nst = [(c + 31) // 32 for c in cnt]
    nmax = nst[0]
    for c in nst[1:]:
        nmax = jnp.maximum(nmax, c)

    def tstep(p, state):
        out = []
        for rr in range(nil):
            t0, t1 = state[rr]
            n0, n1 = _step32(t0, t1, loads[rr], p)
            live = p < nst[rr]
            out.append((jnp.where(live, n0, t0), jnp.where(live, n1, t1)))
        return tuple(out)

    ts = lax.fori_loop(1, nmax, tstep, tuple(ts))
    return [jnp.max(t1) for (_, t1) in ts]


def _topk_kernel(rows_per_w, d2_hbm, thr_hbm, buf0, surv, thrbuf):
    wid = lax.axis_index("s") * NC + lax.axis_index("c")
    base = wid * rows_per_w
    mask0 = lax.iota(jnp.int32, L) == 0
    inf_vec = jnp.full((L,), INF, jnp.float32)
    buf = buf0
    nch = rows_per_w // CHUNK

    def chunk_body(c, carry):
        pltpu.sync_copy(d2_hbm.at[pl.ds(base + c * CHUNK, CHUNK)], buf)

        def rows_body(j, carry2):
            loads = [
                (lambda q, r=j * IL + rr: buf[r, pl.ds(q * L, L)])
                for rr in range(IL)
            ]
            pivots = [_pivot(ld) for ld in loads]
            cnt = [jnp.int32(0)] * IL
            for q in range(0, VPR, 2):
                vs = [(loads[rr](q), loads[rr](q + 1)) for rr in range(IL)]
                ms = [(va <= pivots[rr], vb <= pivots[rr])
                      for rr, (va, vb) in enumerate(vs)]
                cs = [(plsc.all_reduce_population_count(ma)[0],
                       plsc.all_reduce_population_count(mb)[0])
                      for (ma, mb) in ms]
                for rr in range(IL):
                    base_rr = rr * (T + 2 * L)
                    plsc.store_compressed(
                        surv.at[pl.ds(base_rr + cnt[rr], L)],
                        vs[rr][0], mask=ms[rr][0])
                    plsc.store_compressed(
                        surv.at[pl.ds(base_rr + cnt[rr] + cs[rr][0], L)],
                        vs[rr][1], mask=ms[rr][1])
                    cnt[rr] = cnt[rr] + cs[rr][0] + cs[rr][1]
            for rr in range(IL):
                base_rr = rr * (T + 2 * L)
                surv[pl.ds(base_rr + cnt[rr], L)] = inf_vec
                surv[pl.ds(base_rr + cnt[rr] + L, L)] = inf_vec
            thrs = _survivor_thresholds(surv, cnt)
            for rr in range(IL):
                plsc.store_scatter(
                    thrbuf,
                    [jnp.full((L,), c * CHUNK + j * IL + rr, jnp.int32)],
                    jnp.full((L,), thrs[rr], jnp.float32),
                    mask=mask0,
                )
            return carry2

        return lax.fori_loop(0, CHUNK // IL, rows_body, carry)

    lax.fori_loop(0, nch, chunk_body, 0)
    pltpu.sync_copy(thrbuf, thr_hbm.at[pl.ds(base, rows_per_w)])


def _thresholds(d2_flat):
    rows = d2_flat.shape[0]
    rows_per_w = rows // NW
    f = pl.kernel(
        functools.partial(_topk_kernel, rows_per_w),
        out_type=jax.ShapeDtypeStruct((rows,), jnp.float32),
        mesh=plsc.VectorSubcoreMesh(
            core_axis_name="c", subcore_axis_name="s",
            num_cores=NC, num_subcores=NS),
        scratch_types=[
            pltpu.VMEM((CHUNK, T), jnp.float32),
            pltpu.VMEM((IL * (T + 2 * L),), jnp.float32),
            pltpu.VMEM((rows_per_w,), jnp.float32),
        ],
        compiler_params=pltpu.CompilerParams(needs_layout_passes=False),
    )
    return f(d2_flat)


def _smooth_kernel(h_ref, thr_ref, wo_ref, bo_ref, out_ref):
    h = h_ref[0]
    thr = thr_ref[0]
    sq = jnp.sum(h * h, axis=1, keepdims=True)
    g = lax.dot_general(
        h, h, (((1,), (1,)), ((), ())),
        preferred_element_type=jnp.float32,
    )
    d2 = jnp.maximum(sq + jnp.transpose(sq) - 2.0 * g, 0.0)
    row = lax.broadcasted_iota(jnp.int32, (T, T), 0)
    col = lax.broadcasted_iota(jnp.int32, (T, T), 1)
    d2 = jnp.where(row == col, BIG, d2)
    w = jnp.where(d2 <= thr, jnp.exp(d2 * (-1.0 / (2.0 + 1e-8))), 0.0)
    z = jnp.sum(w, axis=1, keepdims=True) + 1e-8
    smooth = lax.dot_general(
        w / z, h, (((1,), (0,)), ((), ())),
        preferred_element_type=jnp.float32,
    )
    lap = jnp.tanh(h - smooth)
    out_ref[0] = lax.dot_general(
        lap, wo_ref[...], (((1,), (1,)), ((), ())),
        preferred_element_type=jnp.float32,
    ) + bo_ref[...]


def _smooth(h, thr, W_out, b_out):
    nb = h.shape[0]
    return pl.pallas_call(
        _smooth_kernel,
        grid=(nb,),
        in_specs=[
            pl.BlockSpec((1, T, H), lambda b: (b, 0, 0)),
            pl.BlockSpec((1, T, 1), lambda b: (b, 0, 0)),
            pl.BlockSpec((H, H), lambda b: (0, 0)),
            pl.BlockSpec((1, H), lambda b: (0, 0)),
        ],
        out_specs=pl.BlockSpec((1, T, H), lambda b: (b, 0, 0)),
        out_shape=jax.ShapeDtypeStruct((nb, T, H), jnp.float32),
    )(h, thr, W_out, b_out.reshape(1, H))


NB_G = 8


@jax.jit
def kernel(x, W_proj, b_proj, W_out, b_out):
    outs = []
    for g in range(B // NB_G):
        xg = x[g * NB_G:(g + 1) * NB_G]
        h, d2 = _distances(xg, W_proj, b_proj)
        thr = _thresholds(d2.reshape(NB_G * T, T))
        outs.append(_smooth(h, thr.reshape(NB_G, T, 1), W_out, b_out))
    return jnp.concatenate(outs, axis=0)


# speedup vs baseline: 1.4406x; 1.1948x over previous
import functools

import jax
import jax.numpy as jnp
from jax import lax
from jax.experimental import pallas as pl
from jax.experimental.pallas import tpu as pltpu
from jax.experimental.pallas import tpu_sc as plsc

B, T, C = 8, 1024, 256
H = 128
K = 32
BIG = 1e9

NC, NS, L = 2, 16, 16
NW = NC * NS
ROWS = B * T
ROWS_PER_W = ROWS // NW
CHUNK = 64
VPR = T // L
IL = 4


def _dist_kernel(x_ref, wp_ref, bp_ref, h_ref, d2_ref):
    x = x_ref[0]
    h = lax.dot_general(
        x, wp_ref[...], (((1,), (1,)), ((), ())),
        preferred_element_type=jnp.float32,
    ) + bp_ref[...]
    h_ref[0] = h
    sq = jnp.sum(h * h, axis=1, keepdims=True)
    g = lax.dot_general(
        h, h, (((1,), (1,)), ((), ())),
        preferred_element_type=jnp.float32,
    )
    d2 = jnp.maximum(sq + jnp.transpose(sq) - 2.0 * g, 0.0)
    row = lax.broadcasted_iota(jnp.int32, (T, T), 0)
    col = lax.broadcasted_iota(jnp.int32, (T, T), 1)
    d2_ref[0] = jnp.where(row == col, BIG, d2)


def _distances(x, W_proj, b_proj):
    nb = x.shape[0]
    return pl.pallas_call(
        _dist_kernel,
        grid=(nb,),
        in_specs=[
            pl.BlockSpec((1, T, C), lambda b: (b, 0, 0)),
            pl.BlockSpec((H, C), lambda b: (0, 0)),
            pl.BlockSpec((1, H), lambda b: (0, 0)),
        ],
        out_specs=[
            pl.BlockSpec((1, T, H), lambda b: (b, 0, 0)),
            pl.BlockSpec((1, T, T), lambda b: (b, 0, 0)),
        ],
        out_shape=[
            jax.ShapeDtypeStruct((nb, T, H), jnp.float32),
            jax.ShapeDtypeStruct((nb, T, T), jnp.float32),
        ],
    )(x, W_proj, b_proj.reshape(1, H))


def _sort_asc(v):
    return plsc.sort_key_val(v, v)[0]


def _sort_desc(v):
    return plsc.sort_key_val(v, v, descending=True)[0]


def _init32(load):
    a = _sort_asc(load(0))
    bd = _sort_desc(load(1))
    lo = jnp.minimum(a, bd)
    hi = jnp.maximum(a, bd)
    return _sort_asc(lo), _sort_asc(hi)


def _step32(t0, t1, load, p):
    a = _sort_asc(load(2 * p))
    bd = _sort_desc(load(2 * p + 1))
    lo = jnp.minimum(a, bd)
    hi = jnp.maximum(a, bd)
    sd0 = _sort_desc(hi)
    sd1 = _sort_desc(lo)
    c0 = jnp.minimum(t0, sd0)
    c1 = jnp.minimum(t1, sd1)
    lo = jnp.minimum(c0, c1)
    hi = jnp.maximum(c0, c1)
    return _sort_asc(lo), _sort_asc(hi)


INF = 3.0e38


def _pivot(load):
    me = load(0)
    mo = load(1)
    for j in range(1, VPR // 2):
        me = jnp.minimum(me, load(2 * j))
        mo = jnp.minimum(mo, load(2 * j + 1))
    return jnp.max(jnp.maximum(me, mo))


def _survivor_thresholds(surv, cnt):
    nil = len(cnt)
    svlen = T + 2 * L
    loads = [(lambda q, rr=rr: surv[pl.ds(rr * svlen + q * L, L)])
             for rr in range(nil)]
    ts = [_init32(ld) for ld in loads]
    nst = [(c + 31) // 32 for c in cnt]
    nmax = nst[0]
    for c in nst[1:]:
        nmax = jnp.maximum(nmax, c)

    def tstep(p, state):
        out = []
        for rr in range(nil):
            t0, t1 = state[rr]
            n0, n1 = _step32(t0, t1, loads[rr], p)
            live = p < nst[rr]
            out.append((jnp.where(live, n0, t0), jnp.where(live, n1, t1)))
        return tuple(out)

    ts = lax.fori_loop(1, nmax, tstep, tuple(ts))
    return [jnp.max(t1) for (_, t1) in ts]


def _topk_kernel(rows_per_w, d2_hbm, thr_hbm, buf0, surv, thrbuf):
    wid = lax.axis_index("s") * NC + lax.axis_index("c")
    base = wid * rows_per_w
    mask0 = lax.iota(jnp.int32, L) == 0
    inf_vec = jnp.full((L,), INF, jnp.float32)
    buf = buf0
    nch = rows_per_w // CHUNK

    def chunk_body(c, carry):
        pltpu.sync_copy(d2_hbm.at[pl.ds(base + c * CHUNK, CHUNK)], buf)

        def rows_body(j, carry2):
            loads = [
                (lambda q, r=j * IL + rr: buf[r, pl.ds(q * L, L)])
                for rr in range(IL)
            ]
            pivots = [_pivot(ld) for ld in loads]
            cnt = [jnp.int32(0)] * IL
            QI = 4
            for q in range(0, VPR, QI):
                vs = [[loads[rr](q + i) for i in range(QI)]
                      for rr in range(IL)]
                ms = [[v <= pivots[rr] for v in vs[rr]] for rr in range(IL)]
                cs = [[plsc.all_reduce_population_count(m)[0]
                       for m in ms[rr]] for rr in range(IL)]
                for rr in range(IL):
                    base_rr = rr * (T + 2 * L)
                    off = cnt[rr]
                    for i in range(QI):
                        plsc.store_compressed(
                            surv.at[pl.ds(base_rr + off, L)],
                            vs[rr][i], mask=ms[rr][i])
                        off = off + cs[rr][i]
                    cnt[rr] = off
            for rr in range(IL):
                base_rr = rr * (T + 2 * L)
                surv[pl.ds(base_rr + cnt[rr], L)] = inf_vec
                surv[pl.ds(base_rr + cnt[rr] + L, L)] = inf_vec
            thrs = _survivor_thresholds(surv, cnt)
            for rr in range(IL):
                plsc.store_scatter(
                    thrbuf,
                    [jnp.full((L,), c * CHUNK + j * IL + rr, jnp.int32)],
                    jnp.full((L,), thrs[rr], jnp.float32),
                    mask=mask0,
                )
            return carry2

        return lax.fori_loop(0, CHUNK // IL, rows_body, carry)

    lax.fori_loop(0, nch, chunk_body, 0)
    pltpu.sync_copy(thrbuf, thr_hbm.at[pl.ds(base, rows_per_w)])


def _thresholds(d2_flat):
    rows = d2_flat.shape[0]
    rows_per_w = rows // NW
    f = pl.kernel(
        functools.partial(_topk_kernel, rows_per_w),
        out_type=jax.ShapeDtypeStruct((rows,), jnp.float32),
        mesh=plsc.VectorSubcoreMesh(
            core_axis_name="c", subcore_axis_name="s",
            num_cores=NC, num_subcores=NS),
        scratch_types=[
            pltpu.VMEM((CHUNK, T), jnp.float32),
            pltpu.VMEM((IL * (T + 2 * L),), jnp.float32),
            pltpu.VMEM((rows_per_w,), jnp.float32),
        ],
        compiler_params=pltpu.CompilerParams(needs_layout_passes=False),
    )
    return f(d2_flat)


def _smooth_kernel(h_ref, thr_ref, wo_ref, bo_ref, out_ref):
    h = h_ref[0]
    thr = thr_ref[0]
    sq = jnp.sum(h * h, axis=1, keepdims=True)
    g = lax.dot_general(
        h, h, (((1,), (1,)), ((), ())),
        preferred_element_type=jnp.float32,
    )
    d2 = jnp.maximum(sq + jnp.transpose(sq) - 2.0 * g, 0.0)
    row = lax.broadcasted_iota(jnp.int32, (T, T), 0)
    col = lax.broadcasted_iota(jnp.int32, (T, T), 1)
    d2 = jnp.where(row == col, BIG, d2)
    w = jnp.where(d2 <= thr, jnp.exp(d2 * (-1.0 / (2.0 + 1e-8))), 0.0)
    z = jnp.sum(w, axis=1, keepdims=True) + 1e-8
    smooth = lax.dot_general(
        w / z, h, (((1,), (0,)), ((), ())),
        preferred_element_type=jnp.float32,
    )
    lap = jnp.tanh(h - smooth)
    out_ref[0] = lax.dot_general(
        lap, wo_ref[...], (((1,), (1,)), ((), ())),
        preferred_element_type=jnp.float32,
    ) + bo_ref[...]


def _smooth(h, thr, W_out, b_out):
    nb = h.shape[0]
    return pl.pallas_call(
        _smooth_kernel,
        grid=(nb,),
        in_specs=[
            pl.BlockSpec((1, T, H), lambda b: (b, 0, 0)),
            pl.BlockSpec((1, T, 1), lambda b: (b, 0, 0)),
            pl.BlockSpec((H, H), lambda b: (0, 0)),
            pl.BlockSpec((1, H), lambda b: (0, 0)),
        ],
        out_specs=pl.BlockSpec((1, T, H), lambda b: (b, 0, 0)),
        out_shape=jax.ShapeDtypeStruct((nb, T, H), jnp.float32),
    )(h, thr, W_out, b_out.reshape(1, H))


NB_G = 8


@jax.jit
def kernel(x, W_proj, b_proj, W_out, b_out):
    outs = []
    for g in range(B // NB_G):
        xg = x[g * NB_G:(g + 1) * NB_G]
        h, d2 = _distances(xg, W_proj, b_proj)
        thr = _thresholds(d2.reshape(NB_G * T, T))
        outs.append(_smooth(h, thr.reshape(NB_G, T, 1), W_out, b_out))
    return jnp.concatenate(outs, axis=0)


# speedup vs baseline: 1.4415x; 1.0006x over previous
import functools

import jax
import jax.numpy as jnp
from jax import lax
from jax.experimental import pallas as pl
from jax.experimental.pallas import tpu as pltpu
from jax.experimental.pallas import tpu_sc as plsc

B, T, C = 8, 1024, 256
H = 128
K = 32
BIG = 1e9

NC, NS, L = 2, 16, 16
NW = NC * NS
ROWS = B * T
ROWS_PER_W = ROWS // NW
CHUNK = 64
VPR = T // L
IL = 4


def _dist_kernel(x_ref, wp_ref, bp_ref, h_ref, d2_ref):
    x = x_ref[0]
    h = lax.dot_general(
        x, wp_ref[...], (((1,), (1,)), ((), ())),
        preferred_element_type=jnp.float32,
    ) + bp_ref[...]
    h_ref[0] = h
    sq = jnp.sum(h * h, axis=1, keepdims=True)
    g = lax.dot_general(
        h, h, (((1,), (1,)), ((), ())),
        preferred_element_type=jnp.float32,
    )
    d2 = jnp.maximum(sq + jnp.transpose(sq) - 2.0 * g, 0.0)
    row = lax.broadcasted_iota(jnp.int32, (T, T), 0)
    col = lax.broadcasted_iota(jnp.int32, (T, T), 1)
    d2_ref[0] = jnp.where(row == col, BIG, d2)


def _distances(x, W_proj, b_proj):
    nb = x.shape[0]
    return pl.pallas_call(
        _dist_kernel,
        grid=(nb,),
        in_specs=[
            pl.BlockSpec((1, T, C), lambda b: (b, 0, 0)),
            pl.BlockSpec((H, C), lambda b: (0, 0)),
            pl.BlockSpec((1, H), lambda b: (0, 0)),
        ],
        out_specs=[
            pl.BlockSpec((1, T, H), lambda b: (b, 0, 0)),
            pl.BlockSpec((1, T, T), lambda b: (b, 0, 0)),
        ],
        out_shape=[
            jax.ShapeDtypeStruct((nb, T, H), jnp.float32),
            jax.ShapeDtypeStruct((nb, T, T), jnp.float32),
        ],
    )(x, W_proj, b_proj.reshape(1, H))


def _sort_asc(v):
    return plsc.sort_key_val(v, v)[0]


def _sort_desc(v):
    return plsc.sort_key_val(v, v, descending=True)[0]


def _init32(load):
    a = _sort_asc(load(0))
    bd = _sort_desc(load(1))
    lo = jnp.minimum(a, bd)
    hi = jnp.maximum(a, bd)
    return _sort_asc(lo), _sort_asc(hi)


def _step32(t0, t1, load, p):
    a = _sort_asc(load(2 * p))
    bd = _sort_desc(load(2 * p + 1))
    lo = jnp.minimum(a, bd)
    hi = jnp.maximum(a, bd)
    sd0 = _sort_desc(hi)
    sd1 = _sort_desc(lo)
    c0 = jnp.minimum(t0, sd0)
    c1 = jnp.minimum(t1, sd1)
    lo = jnp.minimum(c0, c1)
    hi = jnp.maximum(c0, c1)
    return _sort_asc(lo), _sort_asc(hi)


INF = 3.0e38


def _pivot(load):
    me = load(0)
    mo = load(1)
    for j in range(1, VPR // 2):
        me = jnp.minimum(me, load(2 * j))
        mo = jnp.minimum(mo, load(2 * j + 1))
    return jnp.max(jnp.maximum(me, mo))


def _survivor_thresholds(surv, cnt):
    nil = len(cnt)
    svlen = T + 2 * L
    loads = [(lambda q, rr=rr: surv[pl.ds(rr * svlen + q * L, L)])
             for rr in range(nil)]
    ts = [_init32(ld) for ld in loads]
    nst = [(c + 31) // 32 for c in cnt]
    nmax = nst[0]
    for c in nst[1:]:
        nmax = jnp.maximum(nmax, c)

    def tstep(p, state):
        out = []
        for rr in range(nil):
            t0, t1 = state[rr]
            n0, n1 = _step32(t0, t1, loads[rr], p)
            live = p < nst[rr]
            out.append((jnp.where(live, n0, t0), jnp.where(live, n1, t1)))
        return tuple(out)

    ts = lax.fori_loop(1, nmax, tstep, tuple(ts))
    return [jnp.max(t1) for (_, t1) in ts]


def _topk_kernel(rows_per_w, d2_hbm, thr_hbm, buf0, surv, thrbuf):
    wid = lax.axis_index("s") * NC + lax.axis_index("c")
    base = wid * rows_per_w
    mask0 = lax.iota(jnp.int32, L) == 0
    inf_vec = jnp.full((L,), INF, jnp.float32)
    buf = buf0
    nch = rows_per_w // CHUNK

    def chunk_body(c, carry):
        pltpu.sync_copy(d2_hbm.at[pl.ds(base + c * CHUNK, CHUNK)], buf)

        def rows_body(j, carry2):
            loads = [
                (lambda q, r=j * IL + rr: buf[r, pl.ds(q * L, L)])
                for rr in range(IL)
            ]
            pivots = [_pivot(ld) for ld in loads]
            cnt = [jnp.int32(0)] * IL
            QI = 8
            for q in range(0, VPR, QI):
                vs = [[loads[rr](q + i) for i in range(QI)]
                      for rr in range(IL)]
                ms = [[v <= pivots[rr] for v in vs[rr]] for rr in range(IL)]
                cs = [[plsc.all_reduce_population_count(m)[0]
                       for m in ms[rr]] for rr in range(IL)]
                for rr in range(IL):
                    base_rr = rr * (T + 2 * L)
                    off = cnt[rr]
                    for i in range(QI):
                        plsc.store_compressed(
                            surv.at[pl.ds(base_rr + off, L)],
                            vs[rr][i], mask=ms[rr][i])
                        off = off + cs[rr][i]
                    cnt[rr] = off
            for rr in range(IL):
                base_rr = rr * (T + 2 * L)
                surv[pl.ds(base_rr + cnt[rr], L)] = inf_vec
                surv[pl.ds(base_rr + cnt[rr] + L, L)] = inf_vec
            thrs = _survivor_thresholds(surv, cnt)
            for rr in range(IL):
                plsc.store_scatter(
                    thrbuf,
                    [jnp.full((L,), c * CHUNK + j * IL + rr, jnp.int32)],
                    jnp.full((L,), thrs[rr], jnp.float32),
                    mask=mask0,
                )
            return carry2

        return lax.fori_loop(0, CHUNK // IL, rows_body, carry)

    lax.fori_loop(0, nch, chunk_body, 0)
    pltpu.sync_copy(thrbuf, thr_hbm.at[pl.ds(base, rows_per_w)])


def _thresholds(d2_flat):
    rows = d2_flat.shape[0]
    rows_per_w = rows // NW
    f = pl.kernel(
        functools.partial(_topk_kernel, rows_per_w),
        out_type=jax.ShapeDtypeStruct((rows,), jnp.float32),
        mesh=plsc.VectorSubcoreMesh(
            core_axis_name="c", subcore_axis_name="s",
            num_cores=NC, num_subcores=NS),
        scratch_types=[
            pltpu.VMEM((CHUNK, T), jnp.float32),
            pltpu.VMEM((IL * (T + 2 * L),), jnp.float32),
            pltpu.VMEM((rows_per_w,), jnp.float32),
        ],
        compiler_params=pltpu.CompilerParams(needs_layout_passes=False),
    )
    return f(d2_flat)


def _smooth_kernel(h_ref, thr_ref, wo_ref, bo_ref, out_ref):
    h = h_ref[0]
    thr = thr_ref[0]
    sq = jnp.sum(h * h, axis=1, keepdims=True)
    g = lax.dot_general(
        h, h, (((1,), (1,)), ((), ())),
        preferred_element_type=jnp.float32,
    )
    d2 = jnp.maximum(sq + jnp.transpose(sq) - 2.0 * g, 0.0)
    row = lax.broadcasted_iota(jnp.int32, (T, T), 0)
    col = lax.broadcasted_iota(jnp.int32, (T, T), 1)
    d2 = jnp.where(row == col, BIG, d2)
    w = jnp.where(d2 <= thr, jnp.exp(d2 * (-1.0 / (2.0 + 1e-8))), 0.0)
    z = jnp.sum(w, axis=1, keepdims=True) + 1e-8
    smooth = lax.dot_general(
        w / z, h, (((1,), (0,)), ((), ())),
        preferred_element_type=jnp.float32,
    )
    lap = jnp.tanh(h - smooth)
    out_ref[0] = lax.dot_general(
        lap, wo_ref[...], (((1,), (1,)), ((), ())),
        preferred_element_type=jnp.float32,
    ) + bo_ref[...]


def _smooth(h, thr, W_out, b_out):
    nb = h.shape[0]
    return pl.pallas_call(
        _smooth_kernel,
        grid=(nb,),
        in_specs=[
            pl.BlockSpec((1, T, H), lambda b: (b, 0, 0)),
            pl.BlockSpec((1, T, 1), lambda b: (b, 0, 0)),
            pl.BlockSpec((H, H), lambda b: (0, 0)),
            pl.BlockSpec((1, H), lambda b: (0, 0)),
        ],
        out_specs=pl.BlockSpec((1, T, H), lambda b: (b, 0, 0)),
        out_shape=jax.ShapeDtypeStruct((nb, T, H), jnp.float32),
    )(h, thr, W_out, b_out.reshape(1, H))


NB_G = 8


@jax.jit
def kernel(x, W_proj, b_proj, W_out, b_out):
    outs = []
    for g in range(B // NB_G):
        xg = x[g * NB_G:(g + 1) * NB_G]
        h, d2 = _distances(xg, W_proj, b_proj)
        thr = _thresholds(d2.reshape(NB_G * T, T))
        outs.append(_smooth(h, thr.reshape(NB_G, T, 1), W_out, b_out))
    return jnp.concatenate(outs, axis=0)
